# baseline (speedup 1.0000x reference)
import functools

import jax
import jax.numpy as jnp
from jax import lax
from jax.experimental import pallas as pl
from jax.experimental.pallas import tpu as pltpu
from jax.experimental.pallas import tpu_sc as plsc

VOCAB = 10000
EMB = 128
B = 16384
NJ = 4


def _project_body(emb_ref, w_ref, out_ref):
    e = emb_ref[...]
    w = w_ref[...]
    for j in range(NJ):
        wj = w[:, j * EMB:(j + 1) * EMB]
        out_ref[j] = lax.dot_general(
            e, wj, (((1,), (1,)), ((), ())),
            preferred_element_type=jnp.float32)


def _project_table(emb_table, W):
    blk = 2000
    nb = VOCAB // blk
    return pl.pallas_call(
        _project_body,
        grid=(nb,),
        in_specs=[
            pl.BlockSpec((blk, EMB), lambda i: (i, 0)),
            pl.BlockSpec((EMB, NJ * EMB), lambda i: (0, 0)),
        ],
        out_specs=pl.BlockSpec((NJ, blk, EMB), lambda i: (0, i, 0)),
        out_shape=jax.ShapeDtypeStruct((NJ, VOCAB, EMB), jnp.float32),
    )(emb_table, W)


_NC = 2
_NS = 16
_L = 16
_NW = _NC * _NS
_BPW = B // _NW
_CB = 32
_ROWS = _CB * NJ
_NCHUNK = _BPW // _CB


_NBUF = 6


def _sc_body(idx4_hbm, y_hbm, g_hbm, out_hbm, x4_v, rows_v, y_v, acc_v,
             *sems):
    wid = lax.axis_index("s") * _NC + lax.axis_index("c")
    base_b = wid * _BPW
    gsems = sems[:_NBUF]
    ysems = sems[_NBUF:]

    pltpu.sync_copy(idx4_hbm.at[:, pl.ds(base_b, _BPW)], x4_v)

    def fire(t):
        buf = t % _NBUF
        cps = tuple(
            pltpu.async_copy(g_hbm.at[x4_v.at[j, pl.ds(t * _CB, _CB)]],
                             rows_v.at[buf, j], gsems[buf])
            for j in range(NJ))
        return cps + (
            pltpu.async_copy(y_hbm.at[pl.ds(base_b + t * _CB, _CB)],
                             y_v.at[buf], ysems[buf]),
        )

    depth = _NBUF - 1
    pending = [fire(t) for t in range(depth)]
    acc = jnp.zeros((_L,), jnp.float32)
    for t in range(_NCHUNK):
        buf = t % _NBUF
        if t + depth < _NCHUNK:
            pending.append(fire(t + depth))
        for cp in pending.pop(0):
            cp.wait()

        def body(b, acc):
            for s in range(EMB // _L):
                cs = pl.ds(s * _L, _L)
                r = (rows_v[buf, 0, b, cs] + rows_v[buf, 1, b, cs]
                     + rows_v[buf, 2, b, cs] + rows_v[buf, 3, b, cs])
                d = r - y_v[buf, b, cs]
                acc = acc + d * d
            return acc

        acc = lax.fori_loop(0, _CB, body, acc)

    acc_v[...] = acc
    pltpu.sync_copy(acc_v, out_hbm.at[wid])


def _make_sc_kernel():
    mesh = plsc.VectorSubcoreMesh(
        core_axis_name="c", subcore_axis_name="s",
        num_cores=_NC, num_subcores=_NS)
    return functools.partial(
        pl.kernel,
        out_type=jax.ShapeDtypeStruct((_NW, _L), jnp.float32),
        mesh=mesh,
        scratch_types=[
            pltpu.VMEM((NJ, _BPW), jnp.int32),
            pltpu.VMEM((_NBUF, NJ, _CB, EMB), jnp.float32),
            pltpu.VMEM((_NBUF, _CB, EMB), jnp.float32),
            pltpu.VMEM((_L,), jnp.float32),
        ] + [pltpu.SemaphoreType.DMA] * (2 * _NBUF),
        name="cbow_sc",
    )(_sc_body)


_sc_kernel = _make_sc_kernel()


@jax.jit
def kernel(x1, x2, y, emb_table, W):
    g = _project_table(emb_table, W).reshape(NJ * VOCAB, EMB)
    idxs4 = jnp.stack([x1[:, 0], x1[:, 1] + VOCAB,
                       x2[:, 0] + 2 * VOCAB, x2[:, 1] + 3 * VOCAB], axis=0)
    partials = _sc_kernel(idxs4, y, g)
    return jnp.sum(partials) / (B * EMB)


# speedup vs baseline: 1.0190x; 1.0190x over previous
import functools

import jax
import jax.numpy as jnp
from jax import lax
from jax.experimental import pallas as pl
from jax.experimental.pallas import tpu as pltpu
from jax.experimental.pallas import tpu_sc as plsc

VOCAB = 10000
EMB = 128
B = 16384
NJ = 4


def _project_body(emb_ref, w_ref, out_ref):
    e = emb_ref[...]
    w = w_ref[...]
    for j in range(NJ):
        wj = w[:, j * EMB:(j + 1) * EMB]
        out_ref[j] = lax.dot_general(
            e, wj, (((1,), (1,)), ((), ())),
            preferred_element_type=jnp.float32)


def _project_table(emb_table, W):
    blk = 2000
    nb = VOCAB // blk
    return pl.pallas_call(
        _project_body,
        grid=(nb,),
        in_specs=[
            pl.BlockSpec((blk, EMB), lambda i: (i, 0)),
            pl.BlockSpec((EMB, NJ * EMB), lambda i: (0, 0)),
        ],
        out_specs=pl.BlockSpec((NJ, blk, EMB), lambda i: (0, i, 0)),
        out_shape=jax.ShapeDtypeStruct((NJ, VOCAB, EMB), jnp.float32),
    )(emb_table, W)


_NC = 2
_NS = 16
_L = 16
_NW = _NC * _NS
_BPW = B // _NW
_CB = 32
_ROWS = _CB * NJ
_NCHUNK = _BPW // _CB


_NBUF = 4


def _sc_body(idx4_hbm, y_hbm, g_hbm, out_hbm, x4_v, rows_v, y_v, acc_v,
             gsem0, gsem1, gsem2, gsem3, ysem0, ysem1, ysem2, ysem3):
    wid = lax.axis_index("s") * _NC + lax.axis_index("c")
    base_b = wid * _BPW
    gsems = (gsem0, gsem1, gsem2, gsem3)
    ysems = (ysem0, ysem1, ysem2, ysem3)

    pltpu.sync_copy(idx4_hbm.at[:, pl.ds(base_b, _BPW)], x4_v)

    def fire(t):
        buf = t % _NBUF
        cps = tuple(
            pltpu.async_copy(g_hbm.at[x4_v.at[j, pl.ds(t * _CB, _CB)]],
                             rows_v.at[buf, j], gsems[buf])
            for j in range(NJ))
        return cps + (
            pltpu.async_copy(y_hbm.at[pl.ds(base_b + t * _CB, _CB)],
                             y_v.at[buf], ysems[buf]),
        )

    pending = [fire(0), fire(1), fire(2)]
    acc = jnp.zeros((_L,), jnp.float32)
    for t in range(_NCHUNK):
        buf = t % _NBUF
        if t + 3 < _NCHUNK:
            pending.append(fire(t + 3))
        for cp in pending.pop(0):
            cp.wait()

        def body(b, acc):
            for s in range(EMB // _L):
                cs = pl.ds(s * _L, _L)
                r = (rows_v[buf, 0, b, cs] + rows_v[buf, 1, b, cs]
                     + rows_v[buf, 2, b, cs] + rows_v[buf, 3, b, cs])
                d = r - y_v[buf, b, cs]
                acc = acc + d * d
            return acc

        acc = lax.fori_loop(0, _CB, body, acc)

    acc_v[...] = acc
    pltpu.sync_copy(acc_v, out_hbm.at[wid])


def _make_sc_kernel():
    mesh = plsc.VectorSubcoreMesh(
        core_axis_name="c", subcore_axis_name="s",
        num_cores=_NC, num_subcores=_NS)
    return functools.partial(
        pl.kernel,
        out_type=jax.ShapeDtypeStruct((_NW, _L), jnp.float32),
        mesh=mesh,
        scratch_types=[
            pltpu.VMEM((NJ, _BPW), jnp.int32),
            pltpu.VMEM((_NBUF, NJ, _CB, EMB), jnp.float32),
            pltpu.VMEM((_NBUF, _CB, EMB), jnp.float32),
            pltpu.VMEM((_L,), jnp.float32),
            pltpu.SemaphoreType.DMA,
            pltpu.SemaphoreType.DMA,
            pltpu.SemaphoreType.DMA,
            pltpu.SemaphoreType.DMA,
            pltpu.SemaphoreType.DMA,
            pltpu.SemaphoreType.DMA,
            pltpu.SemaphoreType.DMA,
            pltpu.SemaphoreType.DMA,
        ],
        name="cbow_sc",
    )(_sc_body)


_sc_kernel = _make_sc_kernel()


@jax.jit
def kernel(x1, x2, y, emb_table, W):
    g = _project_table(emb_table, W).reshape(NJ * VOCAB, EMB)
    idxs4 = jnp.stack([x1[:, 0], x1[:, 1] + VOCAB,
                       x2[:, 0] + 2 * VOCAB, x2[:, 1] + 3 * VOCAB], axis=0)
    partials = _sc_kernel(idxs4, y, g)
    return jnp.sum(partials) / (B * EMB)
